# Initial kernel scaffold; baseline (speedup 1.0000x reference)
import jax
import jax.numpy as jnp
from jax.experimental import pallas as pl


def kernel(fixed_values, refinable_idx, refinable_params):
    raise NotImplementedError("write your pallas kernel here")


# speedup vs baseline: 49.8092x; 49.8092x over previous
import functools

import jax
import jax.numpy as jnp
from jax import lax
from jax.experimental import pallas as pl
from jax.experimental.pallas import tpu as pltpu
from jax.experimental.pallas import tpu_sc as plsc

N = 16777216
R = 1048576
L = 16
NC, NS = 2, 16
NW = NC * NS
CHUNK = N // NW
P = 32768
PIECES = CHUNK // P
BPP = P // L

_mesh = plsc.VectorSubcoreMesh(core_axis_name="c", subcore_axis_name="s")


@functools.partial(
    pl.kernel,
    mesh=_mesh,
    compiler_params=pltpu.CompilerParams(needs_layout_passes=False),
    out_type=jax.ShapeDtypeStruct((N,), jnp.float32),
    scratch_types=[
        pltpu.VMEM((P,), jnp.float32),
        pltpu.VMEM((BPP,), jnp.int32),
        pltpu.VMEM((BPP,), jnp.float32),
        pltpu.SemaphoreType.DMA,
    ],
)
def _merge(fixed_hbm, idx_hbm, par_hbm, out_hbm, buf_v, idx_v, par_v, sem):
    wid = lax.axis_index("s") * NC + lax.axis_index("c")
    base = wid * CHUNK

    def piece(p, carry):
        pbase = pl.multiple_of(base + p * P, P)
        bbase = pl.multiple_of(pbase // L, BPP)
        cp_fix = pltpu.async_copy(fixed_hbm.at[pl.ds(pbase, P)], buf_v, sem)
        cp_idx = pltpu.async_copy(idx_hbm.at[pl.ds(bbase, BPP)], idx_v, sem)
        cp_par = pltpu.async_copy(par_hbm.at[pl.ds(bbase, BPP)], par_v, sem)
        cp_fix.wait()
        cp_idx.wait()
        cp_par.wait()

        def grp(g, c):
            iv = idx_v[pl.ds(g * L, L)]
            pv = par_v[pl.ds(g * L, L)]
            plsc.store_scatter(buf_v, [iv - pbase], pv)
            return c

        lax.fori_loop(0, BPP // L, grp, 0, unroll=4)
        pltpu.sync_copy(buf_v, out_hbm.at[pl.ds(pbase, P)])
        return carry

    lax.fori_loop(0, PIECES, piece, 0)


def kernel(fixed_values, refinable_idx, refinable_params):
    return _merge(fixed_values, refinable_idx, refinable_params)


# speedup vs baseline: 66.0917x; 1.3269x over previous
import functools

import jax
import jax.numpy as jnp
from jax import lax
from jax.experimental import pallas as pl
from jax.experimental.pallas import tpu as pltpu
from jax.experimental.pallas import tpu_sc as plsc

N = 16777216
R = 1048576
L = 16
NC, NS = 2, 16
NW = NC * NS
CHUNK = N // NW
P = 32768
PIECES = CHUNK // P
BPP = P // L
NBUF = 3

_mesh = plsc.VectorSubcoreMesh(core_axis_name="c", subcore_axis_name="s")

_scratch = (
    [pltpu.VMEM((P,), jnp.float32) for _ in range(NBUF)]
    + [pltpu.VMEM((BPP,), jnp.int32) for _ in range(NBUF)]
    + [pltpu.VMEM((BPP,), jnp.float32) for _ in range(NBUF)]
    + [pltpu.SemaphoreType.DMA for _ in range(2 * NBUF)]
)


@functools.partial(
    pl.kernel,
    mesh=_mesh,
    compiler_params=pltpu.CompilerParams(needs_layout_passes=False),
    out_type=jax.ShapeDtypeStruct((N,), jnp.float32),
    scratch_types=_scratch,
)
def _merge(fixed_hbm, idx_hbm, par_hbm, out_hbm, *scratch):
    bufs = scratch[0:NBUF]
    idxs = scratch[NBUF:2 * NBUF]
    pars = scratch[2 * NBUF:3 * NBUF]
    sin = scratch[3 * NBUF:4 * NBUF]
    sout = scratch[4 * NBUF:5 * NBUF]

    wid = lax.axis_index("s") * NC + lax.axis_index("c")
    base = wid * CHUNK

    def start_in(p):
        b = p % NBUF
        pbase = pl.multiple_of(base + p * P, P)
        bbase = pl.multiple_of(pbase // L, BPP)
        return (
            pltpu.async_copy(fixed_hbm.at[pl.ds(pbase, P)], bufs[b], sin[b]),
            pltpu.async_copy(idx_hbm.at[pl.ds(bbase, BPP)], idxs[b], sin[b]),
            pltpu.async_copy(par_hbm.at[pl.ds(bbase, BPP)], pars[b], sin[b]),
        )

    in_cp = [None] * NBUF
    out_cp = [None] * NBUF
    in_cp[0] = start_in(0)

    for p in range(PIECES):
        b = p % NBUF
        if p + 1 < PIECES:
            b_next = (p + 1) % NBUF
            if p >= NBUF - 1:
                out_cp[b_next].wait()
            in_cp[b_next] = start_in(p + 1)
        for cp in in_cp[b]:
            cp.wait()

        pbase = pl.multiple_of(base + p * P, P)

        def grp(g, c, _b=b, _pbase=pbase):
            iv = idxs[_b][pl.ds(g * L, L)]
            pv = pars[_b][pl.ds(g * L, L)]
            plsc.store_scatter(bufs[_b], [iv - _pbase], pv)
            return c

        lax.fori_loop(0, BPP // L, grp, 0, unroll=4)
        out_cp[b] = pltpu.async_copy(bufs[b], out_hbm.at[pl.ds(pbase, P)],
                                     sout[b])

    for p in range(max(0, PIECES - NBUF), PIECES):
        out_cp[p % NBUF].wait()


def kernel(fixed_values, refinable_idx, refinable_params):
    return _merge(fixed_values, refinable_idx, refinable_params)
